# baseline (speedup 1.0000x reference)
import functools

import jax
import jax.numpy as jnp
from jax.experimental import pallas as pl
from jax.experimental.pallas import tpu as pltpu
from jax.experimental.pallas import tpu_sc as plsc

BATCH = 8


def _fps_body(npoint, xs_ref, ys_ref, zs_ref, nx_ref, ny_ref, nz_ref):
    xs = xs_ref[...]
    ys = ys_ref[...]
    zs = zs_ref[...]
    B, N = xs.shape
    lane = jax.lax.broadcasted_iota(jnp.int32, (B, N), 1)

    def body(i, carry):
        distance, farthest = carry
        mask = lane == farthest
        cx = jnp.sum(jnp.where(mask, xs, 0.0), axis=1, keepdims=True)
        cy = jnp.sum(jnp.where(mask, ys, 0.0), axis=1, keepdims=True)
        cz = jnp.sum(jnp.where(mask, zs, 0.0), axis=1, keepdims=True)
        nx_ref[pl.ds(i, 1), :] = cx.reshape(1, B)
        ny_ref[pl.ds(i, 1), :] = cy.reshape(1, B)
        nz_ref[pl.ds(i, 1), :] = cz.reshape(1, B)
        dx = xs - cx
        dy = ys - cy
        dz = zs - cz
        dist = dx * dx + dy * dy + dz * dz
        distance = jnp.minimum(distance, dist)
        farthest = jnp.argmax(distance, axis=1).astype(jnp.int32).reshape(B, 1)
        return distance, farthest

    init = (jnp.full((B, N), 1e10, jnp.float32), jnp.zeros((B, 1), jnp.int32))
    jax.lax.fori_loop(0, npoint, body, init)


def _fps(xyz, npoint):
    B, N, _ = xyz.shape
    xs, ys, zs = xyz[:, :, 0], xyz[:, :, 1], xyz[:, :, 2]
    out_sd = jax.ShapeDtypeStruct((npoint, B), jnp.float32)
    nx, ny, nz = pl.pallas_call(
        functools.partial(_fps_body, npoint),
        out_shape=(out_sd, out_sd, out_sd),
    )(xs, ys, zs)
    return jnp.stack([nx.T, ny.T, nz.T], axis=-1)


def _minextract(radius2, nsample, n_total, b,
                xs_ref, ys_ref, zs_ref, qx_ref, qy_ref, qz_ref, idx_ref):
    xs = xs_ref[0]
    ys = ys_ref[0]
    zs = zs_ref[0]
    qx = qx_ref[0]
    qy = qy_ref[0]
    qz = qz_ref[0]
    S = qx.shape[0]
    N = xs.shape[1]

    dx = qx - xs
    dy = qy - ys
    dz = qz - zs
    dist = dx * dx + dy * dy + dz * dz
    valid = dist <= radius2

    BIG = 1e9
    col = jax.lax.broadcasted_iota(jnp.int32, (S, N), 1).astype(jnp.float32)
    cand = jnp.where(valid, col, BIG)
    base = jnp.float32(b * n_total)

    m0 = jnp.min(cand, axis=1, keepdims=True)
    idx_ref[0, :, 0:1] = (m0 + base).astype(jnp.int32)
    cand = jnp.where(cand == m0, BIG, cand)
    for s in range(1, nsample):
        m = jnp.min(cand, axis=1, keepdims=True)
        cand = jnp.where(cand == m, BIG, cand)
        m = jnp.where(m >= BIG, m0, m)
        idx_ref[0, :, s:s + 1] = (m + base).astype(jnp.int32)


def _bqidx_body(radius2, nsample, n_total,
                xs_ref, ys_ref, zs_ref, qx_ref, qy_ref, qz_ref, idx_ref):
    b = pl.program_id(0)
    _minextract(radius2, nsample, n_total, b,
                xs_ref, ys_ref, zs_ref, qx_ref, qy_ref, qz_ref, idx_ref)


def _bq_coord_args(xyz, new_xyz):
    B, N, _ = xyz.shape
    S = new_xyz.shape[1]
    xs = xyz[:, :, 0].reshape(B, 1, N)
    ys = xyz[:, :, 1].reshape(B, 1, N)
    zs = xyz[:, :, 2].reshape(B, 1, N)
    qx = new_xyz[:, :, 0:1]
    qy = new_xyz[:, :, 1:2]
    qz = new_xyz[:, :, 2:3]
    rowspec = pl.BlockSpec((1, 1, N), lambda b: (b, 0, 0))
    qspec = pl.BlockSpec((1, S, 1), lambda b: (b, 0, 0))
    specs = [rowspec, rowspec, rowspec, qspec, qspec, qspec]
    return [xs, ys, zs, qx, qy, qz], specs


def _project_body(xyzm_ref, w1_ref, fw_ref):
    fw_ref[...] = jnp.dot(xyzm_ref[...], w1_ref[...],
                          preferred_element_type=jnp.float32)


def _project_fw(xyz, w1):
    B, N, _ = xyz.shape
    D1 = w1.shape[1]
    return pl.pallas_call(
        _project_body,
        out_shape=jax.ShapeDtypeStruct((B * N, D1), jnp.float32),
    )(xyz.reshape(B * N, 3), w1)


def _ballquery_idx(xyz, new_xyz, radius, nsample):
    B, N, _ = xyz.shape
    S = new_xyz.shape[1]
    args, in_specs = _bq_coord_args(xyz, new_xyz)
    return pl.pallas_call(
        functools.partial(_bqidx_body, radius * radius, nsample, N),
        grid=(B,),
        in_specs=in_specs,
        out_specs=pl.BlockSpec((1, S, nsample), lambda b: (b, 0, 0)),
        out_shape=jax.ShapeDtypeStruct((B, S, nsample), jnp.int32),
    )(*args)


def _sc_gather(table, idx, window=128):
    R = idx.shape[0]
    D = table.shape[1]
    idx2 = idx.reshape(1, R)
    mesh = plsc.VectorSubcoreMesh(core_axis_name="c", subcore_axis_name="s")

    @functools.partial(
        pl.kernel,
        out_type=jax.ShapeDtypeStruct((R, D), table.dtype),
        mesh=mesh,
    )
    def gather_kernel(x_hbm, i_hbm, o_hbm):
        def body(i_vmem, o_vmem):
            pltpu.sync_copy(x_hbm.at[i_vmem.at[0]], o_vmem)

        pltpu.emit_pipeline(
            body,
            grid=(R // window,),
            in_specs=[pl.BlockSpec((1, window), index_map=lambda i: (0, i))],
            out_specs=[pl.BlockSpec((window, D), index_map=lambda i: (i, 0))],
            core_axis_name=("c", "s"),
            dimension_semantics=(pltpu.PARALLEL,),
        )(i_hbm, o_hbm)

    return gather_kernel(table, idx2)


def _mlpmax_body(nsample, has_next, g_ref, nxyz_ref, w1a_ref, b1_ref, w2_ref,
                 b2_ref, w3_ref, b3_ref, *rest):
    if has_next:
        w1an_ref, w1bn_ref, out_ref, fwn_ref = rest
    else:
        (out_ref,) = rest
    g = g_ref[...]
    Qns, D1 = g.shape
    Q = Qns // nsample
    nxyz = nxyz_ref[...]
    cq = jnp.dot(nxyz, w1a_ref[...], preferred_element_type=jnp.float32)
    h = g.reshape(Q, nsample, D1) - cq[:, None, :] + b1_ref[...][None, None, :]
    h1 = jnp.maximum(h, 0.0).reshape(Qns, D1)
    h2 = jnp.maximum(
        jnp.dot(h1, w2_ref[...], preferred_element_type=jnp.float32)
        + b2_ref[...][None, :], 0.0)
    h3 = jnp.maximum(
        jnp.dot(h2, w3_ref[...], preferred_element_type=jnp.float32)
        + b3_ref[...][None, :], 0.0)
    D3 = h3.shape[1]
    pooled = jnp.max(h3.reshape(Q, nsample, D3), axis=1)
    out_ref[...] = pooled
    if has_next:
        fwn_ref[...] = (
            jnp.dot(nxyz, w1an_ref[...], preferred_element_type=jnp.float32)
            + jnp.dot(pooled, w1bn_ref[...],
                      preferred_element_type=jnp.float32))


def _mlpmax(g, new_xyz, nsample, w1a, b1, w2, b2, w3, b3, q_block,
            w1_next=None):
    B, S, _ = new_xyz.shape
    D1 = g.shape[1]
    D3 = w3.shape[1]
    R = B * S
    nxyz_flat = new_xyz.reshape(R, 3)
    grid = (R // q_block,)
    has_next = w1_next is not None

    def whole(a):
        return pl.BlockSpec(a.shape, lambda i: tuple(0 for _ in a.shape))

    in_specs = [
        pl.BlockSpec((q_block * nsample, D1), lambda i: (i, 0)),
        pl.BlockSpec((q_block, 3), lambda i: (i, 0)),
        whole(w1a), whole(b1), whole(w2), whole(b2), whole(w3), whole(b3),
    ]
    args = [g, nxyz_flat, w1a, b1, w2, b2, w3, b3]
    out_specs = pl.BlockSpec((q_block, D3), lambda i: (i, 0))
    out_shape = jax.ShapeDtypeStruct((R, D3), jnp.float32)
    if has_next:
        w1an, w1bn = w1_next[0:3, :], w1_next[3:, :]
        in_specs += [whole(w1an), whole(w1bn)]
        args += [w1an, w1bn]
        D1n = w1_next.shape[1]
        out_specs = (out_specs, pl.BlockSpec((q_block, D1n), lambda i: (i, 0)))
        out_shape = (out_shape, jax.ShapeDtypeStruct((R, D1n), jnp.float32))

    out = pl.pallas_call(
        functools.partial(_mlpmax_body, nsample, has_next),
        grid=grid,
        in_specs=in_specs,
        out_specs=out_specs,
        out_shape=out_shape,
    )(*args)
    if has_next:
        pooled, fwn = out
        return pooled.reshape(B, S, D3), fwn
    return out.reshape(B, S, D3)


def _sa4_body(xyz_ref, pts_ref, w1a_ref, w1b_ref, b1_ref, w2_ref, b2_ref,
              w3_ref, b3_ref, out_ref):
    x = xyz_ref[0]
    p = pts_ref[0]
    h1 = jnp.maximum(
        jnp.dot(x, w1a_ref[...], preferred_element_type=jnp.float32)
        + jnp.dot(p, w1b_ref[...], preferred_element_type=jnp.float32)
        + b1_ref[...][None, :], 0.0)
    h2 = jnp.maximum(
        jnp.dot(h1, w2_ref[...], preferred_element_type=jnp.float32)
        + b2_ref[...][None, :], 0.0)
    h3 = jnp.maximum(
        jnp.dot(h2, w3_ref[...], preferred_element_type=jnp.float32)
        + b3_ref[...][None, :], 0.0)
    out_ref[0] = jnp.max(h3, axis=0, keepdims=True)


def _sa4(xyz, pts, w1, b1, w2, b2, w3, b3):
    B, M, _ = xyz.shape
    C = pts.shape[2]
    D3 = w3.shape[1]
    w1a, w1b = w1[0:3, :], w1[3:, :]

    def whole(a):
        return pl.BlockSpec(a.shape, lambda b: tuple(0 for _ in a.shape))

    out = pl.pallas_call(
        _sa4_body,
        grid=(B,),
        in_specs=[
            pl.BlockSpec((1, M, 3), lambda b: (b, 0, 0)),
            pl.BlockSpec((1, M, C), lambda b: (b, 0, 0)),
            whole(w1a), whole(w1b), whole(b1), whole(w2), whole(b2),
            whole(w3), whole(b3),
        ],
        out_specs=pl.BlockSpec((1, 1, D3), lambda b: (b, 0, 0)),
        out_shape=jax.ShapeDtypeStruct((B, 1, D3), jnp.float32),
    )(xyz, pts, w1a, w1b, b1, w2, b2, w3, b3)
    return out.reshape(B, D3)


def _pad_layer1(w1, b1, w2, to=128):
    d1 = w1.shape[1]
    if d1 % to == 0:
        return w1, b1, w2
    pad = to - d1 % to
    w1 = jnp.concatenate([w1, jnp.zeros((w1.shape[0], pad), w1.dtype)], axis=1)
    b1 = jnp.concatenate([b1, jnp.zeros((pad,), b1.dtype)])
    w2 = jnp.concatenate([w2, jnp.zeros((pad, w2.shape[1]), w2.dtype)], axis=0)
    return w1, b1, w2


def kernel(input, params):
    xyz = input
    B, N, _ = xyz.shape
    (w11, b11), (w12, b12), (w13, b13) = params["sa1"]
    (w21, b21), (w22, b22), (w23, b23) = params["sa2"]
    (w31, b31), (w32, b32), (w33, b33) = params["sa3"]
    w11, b11, w12 = _pad_layer1(w11, b11, w12)
    w21, b21, w22 = _pad_layer1(w21, b21, w22)
    w31, b31, w32 = _pad_layer1(w31, b31, w32)

    new1 = _fps(xyz, 1024)
    new2 = new1[:, :256]
    new3 = new1[:, :64]

    fw1 = _project_fw(xyz, w11)
    half = 512
    idx1a = _ballquery_idx(xyz, new1[:, :half], 0.1, 32)
    g1a = _sc_gather(fw1, idx1a.reshape(-1))
    idx1b = _ballquery_idx(xyz, new1[:, half:], 0.1, 32)
    g1b = _sc_gather(fw1, idx1b.reshape(-1))
    idx2 = _ballquery_idx(new1, new2, 0.2, 32)
    idx3 = _ballquery_idx(new2, new3, 0.4, 64)

    D1n = w21.shape[1]
    _, fw2a = _mlpmax(g1a, new1[:, :half], 32, w11[0:3, :], b11, w12, b12,
                      w13, b13, 512, w1_next=w21)
    _, fw2b = _mlpmax(g1b, new1[:, half:], 32, w11[0:3, :], b11, w12, b12,
                      w13, b13, 512, w1_next=w21)
    fw2 = jnp.concatenate(
        [fw2a.reshape(B, half, D1n), fw2b.reshape(B, half, D1n)],
        axis=1).reshape(B * 1024, D1n)
    g2a = _sc_gather(fw2, idx2[:, :128].reshape(-1))
    g2b = _sc_gather(fw2, idx2[:, 128:].reshape(-1))
    D1n3 = w31.shape[1]
    _, fw3a = _mlpmax(g2a, new2[:, :128], 32, w21[0:3, :], b21, w22, b22,
                      w23, b23, 128, w1_next=w31)
    _, fw3b = _mlpmax(g2b, new2[:, 128:], 32, w21[0:3, :], b21, w22, b22,
                      w23, b23, 128, w1_next=w31)
    fw3 = jnp.concatenate(
        [fw3a.reshape(B, 128, D1n3), fw3b.reshape(B, 128, D1n3)],
        axis=1).reshape(B * 256, D1n3)
    g3a = _sc_gather(fw3, idx3[:, :32].reshape(-1))
    g3b = _sc_gather(fw3, idx3[:, 32:].reshape(-1))
    pts3a = _mlpmax(g3a, new3[:, :32], 64, w31[0:3, :], b31, w32, b32,
                    w33, b33, 32)
    pts3b = _mlpmax(g3b, new3[:, 32:], 64, w31[0:3, :], b31, w32, b32,
                    w33, b33, 32)
    pts3 = jnp.concatenate([pts3a, pts3b], axis=1)

    (w1, b1), (w2, b2), (w3, b3) = params["sa4"]
    return _sa4(new3, pts3, w1, b1, w2, b2, w3, b3)


# speedup vs baseline: 1.0163x; 1.0163x over previous
import functools

import jax
import jax.numpy as jnp
from jax.experimental import pallas as pl
from jax.experimental.pallas import tpu as pltpu
from jax.experimental.pallas import tpu_sc as plsc

BATCH = 8


def _fps_body(npoint, xs_ref, ys_ref, zs_ref, nx_ref, ny_ref, nz_ref):
    xs = xs_ref[...]
    ys = ys_ref[...]
    zs = zs_ref[...]
    B, N = xs.shape
    lane = jax.lax.broadcasted_iota(jnp.int32, (B, N), 1)

    def body(i, carry):
        distance, farthest = carry
        mask = lane == farthest
        cx = jnp.sum(jnp.where(mask, xs, 0.0), axis=1, keepdims=True)
        cy = jnp.sum(jnp.where(mask, ys, 0.0), axis=1, keepdims=True)
        cz = jnp.sum(jnp.where(mask, zs, 0.0), axis=1, keepdims=True)
        nx_ref[pl.ds(i, 1), :] = cx.reshape(1, B)
        ny_ref[pl.ds(i, 1), :] = cy.reshape(1, B)
        nz_ref[pl.ds(i, 1), :] = cz.reshape(1, B)
        dx = xs - cx
        dy = ys - cy
        dz = zs - cz
        dist = dx * dx + dy * dy + dz * dz
        distance = jnp.minimum(distance, dist)
        farthest = jnp.argmax(distance, axis=1).astype(jnp.int32).reshape(B, 1)
        return distance, farthest

    init = (jnp.full((B, N), 1e10, jnp.float32), jnp.zeros((B, 1), jnp.int32))
    jax.lax.fori_loop(0, npoint, body, init)


def _fps(xyz, npoint):
    B, N, _ = xyz.shape
    xs, ys, zs = xyz[:, :, 0], xyz[:, :, 1], xyz[:, :, 2]
    out_sd = jax.ShapeDtypeStruct((npoint, B), jnp.float32)
    nx, ny, nz = pl.pallas_call(
        functools.partial(_fps_body, npoint),
        out_shape=(out_sd, out_sd, out_sd),
    )(xs, ys, zs)
    return jnp.stack([nx.T, ny.T, nz.T], axis=-1)


def _minextract(radius2, nsample, n_total, b,
                xs_ref, ys_ref, zs_ref, qx_ref, qy_ref, qz_ref, idx_ref):
    xs = xs_ref[0]
    ys = ys_ref[0]
    zs = zs_ref[0]
    qx = qx_ref[0]
    qy = qy_ref[0]
    qz = qz_ref[0]
    S = qx.shape[0]
    N = xs.shape[1]

    dx = qx - xs
    dy = qy - ys
    dz = qz - zs
    dist = dx * dx + dy * dy + dz * dz
    valid = dist <= radius2

    BIG = 1e9
    col = jax.lax.broadcasted_iota(jnp.int32, (S, N), 1).astype(jnp.float32)
    cand = jnp.where(valid, col, BIG)
    base = jnp.float32(b * n_total)

    m0 = jnp.min(cand, axis=1, keepdims=True)
    idx_ref[0, :, 0:1] = (m0 + base).astype(jnp.int32)
    cand = jnp.where(cand == m0, BIG, cand)
    for s in range(1, nsample):
        m = jnp.min(cand, axis=1, keepdims=True)
        cand = jnp.where(cand == m, BIG, cand)
        m = jnp.where(m >= BIG, m0, m)
        idx_ref[0, :, s:s + 1] = (m + base).astype(jnp.int32)


def _bqidx_body(radius2, nsample, n_total,
                xs_ref, ys_ref, zs_ref, qx_ref, qy_ref, qz_ref, idx_ref):
    b = pl.program_id(0)
    _minextract(radius2, nsample, n_total, b,
                xs_ref, ys_ref, zs_ref, qx_ref, qy_ref, qz_ref, idx_ref)


def _bq_coord_args(xyz, new_xyz):
    B, N, _ = xyz.shape
    S = new_xyz.shape[1]
    xs = xyz[:, :, 0].reshape(B, 1, N)
    ys = xyz[:, :, 1].reshape(B, 1, N)
    zs = xyz[:, :, 2].reshape(B, 1, N)
    qx = new_xyz[:, :, 0:1]
    qy = new_xyz[:, :, 1:2]
    qz = new_xyz[:, :, 2:3]
    rowspec = pl.BlockSpec((1, 1, N), lambda b: (b, 0, 0))
    qspec = pl.BlockSpec((1, S, 1), lambda b: (b, 0, 0))
    specs = [rowspec, rowspec, rowspec, qspec, qspec, qspec]
    return [xs, ys, zs, qx, qy, qz], specs


def _project_body(xyzm_ref, w1_ref, fw_ref):
    fw_ref[...] = jnp.dot(xyzm_ref[...], w1_ref[...],
                          preferred_element_type=jnp.float32)


def _project_fw(xyz, w1):
    B, N, _ = xyz.shape
    D1 = w1.shape[1]
    return pl.pallas_call(
        _project_body,
        out_shape=jax.ShapeDtypeStruct((B * N, D1), jnp.float32),
    )(xyz.reshape(B * N, 3), w1)


def _ballquery_idx(xyz, new_xyz, radius, nsample):
    B, N, _ = xyz.shape
    S = new_xyz.shape[1]
    args, in_specs = _bq_coord_args(xyz, new_xyz)
    return pl.pallas_call(
        functools.partial(_bqidx_body, radius * radius, nsample, N),
        grid=(B,),
        in_specs=in_specs,
        out_specs=pl.BlockSpec((1, S, nsample), lambda b: (b, 0, 0)),
        out_shape=jax.ShapeDtypeStruct((B, S, nsample), jnp.int32),
    )(*args)


def _sc_gather(table, idx, window=128):
    R = idx.shape[0]
    D = table.shape[1]
    idx2 = idx.reshape(1, R)
    mesh = plsc.VectorSubcoreMesh(core_axis_name="c", subcore_axis_name="s")

    @functools.partial(
        pl.kernel,
        out_type=jax.ShapeDtypeStruct((R, D), table.dtype),
        mesh=mesh,
    )
    def gather_kernel(x_hbm, i_hbm, o_hbm):
        def body(i_vmem, o_vmem):
            pltpu.sync_copy(x_hbm.at[i_vmem.at[0]], o_vmem)

        pltpu.emit_pipeline(
            body,
            grid=(R // window,),
            in_specs=[pl.BlockSpec((1, window), index_map=lambda i: (0, i))],
            out_specs=[pl.BlockSpec((window, D), index_map=lambda i: (i, 0))],
            core_axis_name=("c", "s"),
            dimension_semantics=(pltpu.PARALLEL,),
        )(i_hbm, o_hbm)

    return gather_kernel(table, idx2)


def _mlpmax_body(nsample, has_next, g_ref, nxyz_ref, w1a_ref, b1_ref, w2_ref,
                 b2_ref, w3_ref, b3_ref, *rest):
    if has_next:
        w1an_ref, w1bn_ref, out_ref, fwn_ref = rest
    else:
        (out_ref,) = rest
    g = g_ref[...]
    Qns, D1 = g.shape
    Q = Qns // nsample
    nxyz = nxyz_ref[...]
    cq = jnp.dot(nxyz, w1a_ref[...], preferred_element_type=jnp.float32)
    h = g.reshape(Q, nsample, D1) - cq[:, None, :] + b1_ref[...][None, None, :]
    h1 = jnp.maximum(h, 0.0).reshape(Qns, D1)
    h2 = jnp.maximum(
        jnp.dot(h1, w2_ref[...], preferred_element_type=jnp.float32)
        + b2_ref[...][None, :], 0.0)
    h3 = jnp.maximum(
        jnp.dot(h2, w3_ref[...], preferred_element_type=jnp.float32)
        + b3_ref[...][None, :], 0.0)
    D3 = h3.shape[1]
    pooled = jnp.max(h3.reshape(Q, nsample, D3), axis=1)
    out_ref[...] = pooled
    if has_next:
        fwn_ref[...] = (
            jnp.dot(nxyz, w1an_ref[...], preferred_element_type=jnp.float32)
            + jnp.dot(pooled, w1bn_ref[...],
                      preferred_element_type=jnp.float32))


def _mlpmax(g, new_xyz, nsample, w1a, b1, w2, b2, w3, b3, q_block,
            w1_next=None):
    B, S, _ = new_xyz.shape
    D1 = g.shape[1]
    D3 = w3.shape[1]
    R = B * S
    nxyz_flat = new_xyz.reshape(R, 3)
    grid = (R // q_block,)
    has_next = w1_next is not None

    def whole(a):
        return pl.BlockSpec(a.shape, lambda i: tuple(0 for _ in a.shape))

    in_specs = [
        pl.BlockSpec((q_block * nsample, D1), lambda i: (i, 0)),
        pl.BlockSpec((q_block, 3), lambda i: (i, 0)),
        whole(w1a), whole(b1), whole(w2), whole(b2), whole(w3), whole(b3),
    ]
    args = [g, nxyz_flat, w1a, b1, w2, b2, w3, b3]
    out_specs = pl.BlockSpec((q_block, D3), lambda i: (i, 0))
    out_shape = jax.ShapeDtypeStruct((R, D3), jnp.float32)
    if has_next:
        w1an, w1bn = w1_next[0:3, :], w1_next[3:, :]
        in_specs += [whole(w1an), whole(w1bn)]
        args += [w1an, w1bn]
        D1n = w1_next.shape[1]
        out_specs = (out_specs, pl.BlockSpec((q_block, D1n), lambda i: (i, 0)))
        out_shape = (out_shape, jax.ShapeDtypeStruct((R, D1n), jnp.float32))

    out = pl.pallas_call(
        functools.partial(_mlpmax_body, nsample, has_next),
        grid=grid,
        in_specs=in_specs,
        out_specs=out_specs,
        out_shape=out_shape,
    )(*args)
    if has_next:
        pooled, fwn = out
        return pooled.reshape(B, S, D3), fwn
    return out.reshape(B, S, D3)


def _sa4_body(xyz_ref, pts_ref, w1a_ref, w1b_ref, b1_ref, w2_ref, b2_ref,
              w3_ref, b3_ref, out_ref):
    x = xyz_ref[0]
    p = pts_ref[0]
    h1 = jnp.maximum(
        jnp.dot(x, w1a_ref[...], preferred_element_type=jnp.float32)
        + jnp.dot(p, w1b_ref[...], preferred_element_type=jnp.float32)
        + b1_ref[...][None, :], 0.0)
    h2 = jnp.maximum(
        jnp.dot(h1, w2_ref[...], preferred_element_type=jnp.float32)
        + b2_ref[...][None, :], 0.0)
    h3 = jnp.maximum(
        jnp.dot(h2, w3_ref[...], preferred_element_type=jnp.float32)
        + b3_ref[...][None, :], 0.0)
    out_ref[0] = jnp.max(h3, axis=0, keepdims=True)


def _sa4(xyz, pts, w1, b1, w2, b2, w3, b3):
    B, M, _ = xyz.shape
    C = pts.shape[2]
    D3 = w3.shape[1]
    w1a, w1b = w1[0:3, :], w1[3:, :]

    def whole(a):
        return pl.BlockSpec(a.shape, lambda b: tuple(0 for _ in a.shape))

    out = pl.pallas_call(
        _sa4_body,
        grid=(B,),
        in_specs=[
            pl.BlockSpec((1, M, 3), lambda b: (b, 0, 0)),
            pl.BlockSpec((1, M, C), lambda b: (b, 0, 0)),
            whole(w1a), whole(w1b), whole(b1), whole(w2), whole(b2),
            whole(w3), whole(b3),
        ],
        out_specs=pl.BlockSpec((1, 1, D3), lambda b: (b, 0, 0)),
        out_shape=jax.ShapeDtypeStruct((B, 1, D3), jnp.float32),
    )(xyz, pts, w1a, w1b, b1, w2, b2, w3, b3)
    return out.reshape(B, D3)


def _pad_layer1(w1, b1, w2, to=128):
    d1 = w1.shape[1]
    if d1 % to == 0:
        return w1, b1, w2
    pad = to - d1 % to
    w1 = jnp.concatenate([w1, jnp.zeros((w1.shape[0], pad), w1.dtype)], axis=1)
    b1 = jnp.concatenate([b1, jnp.zeros((pad,), b1.dtype)])
    w2 = jnp.concatenate([w2, jnp.zeros((pad, w2.shape[1]), w2.dtype)], axis=0)
    return w1, b1, w2


def kernel(input, params):
    xyz = input
    B, N, _ = xyz.shape
    (w11, b11), (w12, b12), (w13, b13) = params["sa1"]
    (w21, b21), (w22, b22), (w23, b23) = params["sa2"]
    (w31, b31), (w32, b32), (w33, b33) = params["sa3"]
    w11, b11, w12 = _pad_layer1(w11, b11, w12)
    w21, b21, w22 = _pad_layer1(w21, b21, w22)
    w31, b31, w32 = _pad_layer1(w31, b31, w32)

    new1 = _fps(xyz, 1024)
    new2 = new1[:, :256]
    new3 = new1[:, :64]

    fw1 = _project_fw(xyz, w11)
    half = 512
    idx1a = _ballquery_idx(xyz, new1[:, :half], 0.1, 32)
    g1a = _sc_gather(fw1, idx1a.reshape(-1))
    idx1b = _ballquery_idx(xyz, new1[:, half:], 0.1, 32)
    g1b = _sc_gather(fw1, idx1b.reshape(-1))
    idx2 = _ballquery_idx(new1, new2, 0.2, 32)
    idx3 = _ballquery_idx(new2, new3, 0.4, 64)

    D1n = w21.shape[1]
    _, fw2a = _mlpmax(g1a, new1[:, :half], 32, w11[0:3, :], b11, w12, b12,
                      w13, b13, 512, w1_next=w21)
    _, fw2b = _mlpmax(g1b, new1[:, half:], 32, w11[0:3, :], b11, w12, b12,
                      w13, b13, 512, w1_next=w21)
    fw2 = jnp.concatenate(
        [fw2a.reshape(B, half, D1n), fw2b.reshape(B, half, D1n)],
        axis=1).reshape(B * 1024, D1n)
    g2 = _sc_gather(fw2, idx2.reshape(-1))
    _, fw3 = _mlpmax(g2, new2, 32, w21[0:3, :], b21, w22, b22, w23, b23, 256,
                     w1_next=w31)
    g3 = _sc_gather(fw3, idx3.reshape(-1))
    pts3 = _mlpmax(g3, new3, 64, w31[0:3, :], b31, w32, b32, w33, b33, 64)

    (w1, b1), (w2, b2), (w3, b3) = params["sa4"]
    return _sa4(new3, pts3, w1, b1, w2, b2, w3, b3)
